# Initial kernel scaffold; baseline (speedup 1.0000x reference)
import jax
import jax.numpy as jnp
from jax.experimental import pallas as pl


def kernel(xyz, p1, feat, W1, b1, g1, be1, W2, b2, g2, be2):
    raise NotImplementedError("write your pallas kernel here")


# speedup vs baseline: 11.8295x; 11.8295x over previous
import functools

import jax
import jax.numpy as jnp
from jax import lax
from jax.experimental import pallas as pl
from jax.experimental.pallas import tpu as pltpu
from jax.experimental.pallas import tpu_sc as plsc

_B = 2
_N1 = 1024
_N2 = 16384
_C = 128
_K = 50
_NW = 32
_RPW = (_B * _N1) // _NW
_SPB = _N1 // _RPW
_NG = _N2 // 16
_NS = _NG // 16
_RK = _RPW * _K
_INF = 3.0e38
_EPS = 1e-5
_NTOT = float(_B * _N1 * _K)
_ROWBLK = 512
_NBLK = (_B * _N1 * _K) // _ROWBLK


def _full16(v, dtype=jnp.int32):
    return jnp.full((16,), v, dtype)


def _knn_body(xp, yp, zp, sp, qx, qy, qz, qs, gidx,
              xb, yb, zb, sb, qxb, qyb, qzb, qsb, db, cmb, smb, ixb):
    wid = lax.axis_index("s") * 2 + lax.axis_index("c")
    b = wid // _SPB
    qoff = (wid % _SPB) * _RPW
    pltpu.sync_copy(xp.at[b], xb)
    pltpu.sync_copy(yp.at[b], yb)
    pltpu.sync_copy(zp.at[b], zb)
    pltpu.sync_copy(sp.at[b], sb)
    pltpu.sync_copy(qx.at[b, pl.ds(qoff, _RPW)], qxb)
    pltpu.sync_copy(qy.at[b, pl.ds(qoff, _RPW)], qyb)
    pltpu.sync_copy(qz.at[b, pl.ds(qoff, _RPW)], qzb)
    pltpu.sync_copy(qs.at[b, pl.ds(qoff, _RPW)], qsb)
    iota16 = lax.iota(jnp.int32, 16)
    batch_base = b * _N2

    def qbody(q, _):
        qi = _full16(q)
        qxs = plsc.load_gather(qxb, [qi])
        qys = plsc.load_gather(qyb, [qi])
        qzs = plsc.load_gather(qzb, [qi])
        qss = plsc.load_gather(qsb, [qi])

        for u in range(4):
            def vbody(v2, smacc, u=u):
                base = (u + 4 * v2) * 16
                gm = jnp.full((16,), _INF)
                for m in range(16):
                    off = m * _NG + base
                    xv = xb[pl.ds(off, 16)]
                    yv = yb[pl.ds(off, 16)]
                    zv = zb[pl.ds(off, 16)]
                    sv = sb[pl.ds(off, 16)]
                    d = (qss + sv) - 2.0 * ((qxs * xv + qys * yv)
                                            + qzs * zv)
                    db[pl.ds(off, 16)] = d
                    gm = jnp.minimum(gm, d)
                cmb[pl.ds(base, 16)] = gm
                return jnp.minimum(smacc, gm)
            smacc = lax.fori_loop(0, 16, vbody, jnp.full((16,), _INF))
            smb[pl.ds(u * 16, 16)] = smacc

        rowbase = q * _K
        for vec_i in range(4):
            nr = 16 if vec_i < 3 else 2

            def rbody(r2, outvec):
                rm = jnp.full((16,), _INF)
                ra = jnp.zeros((16,), jnp.int32)
                for i in range(4):
                    smv = smb[pl.ds(i * 16, 16)]
                    msk = smv < rm
                    rm = jnp.where(msk, smv, rm)
                    ra = jnp.where(msk, jnp.full((16,), i, jnp.int32), ra)
                m = jnp.min(rm)
                key = jnp.where(rm == m, ra * 16 + iota16,
                                jnp.full((16,), 1024, jnp.int32))
                s_star = jnp.min(key)
                cmg = plsc.load_gather(cmb, [iota16 * _NS + s_star])
                m2_s = jnp.max(plsc.all_reduce_ffs(cmg == m))
                g_star = m2_s * _NS + s_star
                dg = plsc.load_gather(db, [iota16 * _NG + g_star])
                mm_s = jnp.max(plsc.all_reduce_ffs(dg == m))
                j_loc = mm_s * _NG + g_star
                j_cand = s_star * 256 + m2_s * 16 + mm_s
                outvec = jnp.where(iota16 == r2,
                                   _full16(j_cand + batch_base), outvec)
                lane0 = iota16 == 0
                plsc.store_scatter(db, [_full16(j_loc)],
                                   jnp.full((16,), _INF), mask=lane0)
                ncm = jnp.min(jnp.where(iota16 == mm_s, _INF, dg))
                plsc.store_scatter(cmb, [_full16(g_star)],
                                   jnp.full((16,), ncm), mask=lane0)
                nsm = jnp.min(jnp.where(iota16 == m2_s, ncm, cmg))
                plsc.store_scatter(smb, [_full16(s_star)],
                                   jnp.full((16,), nsm), mask=lane0)
                return outvec

            outvec = lax.fori_loop(0, nr, rbody, jnp.zeros((16,), jnp.int32))
            ixb[pl.ds(rowbase + vec_i * 16, 16)] = outvec
        return 0

    lax.fori_loop(0, _RPW, qbody, 0)
    pltpu.sync_copy(ixb.at[pl.ds(0, _RK)], gidx.at[pl.ds(wid * _RK, _RK)])


def _knn_call(xp, yp, zp, sp, qx, qy, qz, qs):
    return pl.kernel(
        _knn_body,
        out_type=jax.ShapeDtypeStruct((_B * _N1 * _K,), jnp.int32),
        mesh=plsc.VectorSubcoreMesh(core_axis_name="c", subcore_axis_name="s"),
        compiler_params=pltpu.CompilerParams(needs_layout_passes=False),
        scratch_types=[
            pltpu.VMEM((_N2,), jnp.float32),
            pltpu.VMEM((_N2,), jnp.float32),
            pltpu.VMEM((_N2,), jnp.float32),
            pltpu.VMEM((_N2,), jnp.float32),
            pltpu.VMEM((_RPW,), jnp.float32),
            pltpu.VMEM((_RPW,), jnp.float32),
            pltpu.VMEM((_RPW,), jnp.float32),
            pltpu.VMEM((_RPW,), jnp.float32),
            pltpu.VMEM((_N2,), jnp.float32),
            pltpu.VMEM((_NG,), jnp.float32),
            pltpu.VMEM((_NS,), jnp.float32),
            pltpu.VMEM((_RK + 16,), jnp.int32),
        ],
    )(xp, yp, zp, sp, qx, qy, qz, qs)


def _gather_body(feat, gidx, out, idxb, rowb, sem):
    wid = lax.axis_index("s") * 2 + lax.axis_index("c")
    nchunks = _RK // 128
    pltpu.sync_copy(gidx.at[pl.ds(wid * _RK, _RK)], idxb)

    def cbody(c, _):
        pltpu.async_copy(feat.at[idxb.at[pl.ds(c * 128, 128)]], rowb, sem).wait()
        pltpu.sync_copy(rowb, out.at[pl.ds(wid * _RK + c * 128, 128)])
        return 0

    lax.fori_loop(0, nchunks, cbody, 0)


def _gather_call(featflat, gidx):
    return pl.kernel(
        _gather_body,
        out_type=jax.ShapeDtypeStruct((_B * _N1 * _K, _C), jnp.float32),
        mesh=plsc.VectorSubcoreMesh(core_axis_name="c", subcore_axis_name="s"),
        compiler_params=pltpu.CompilerParams(needs_layout_passes=False),
        scratch_types=[
            pltpu.VMEM((_RK,), jnp.int32),
            pltpu.VMEM((128, _C), jnp.float32),
            pltpu.SemaphoreType.DMA,
        ],
    )(featflat, gidx)


def _fold_stats(s_acc, m_acc, w, wt, bias, gamma, beta):
    mu = s_acc / _NTOT
    mean = jnp.dot(mu, wt, preferred_element_type=jnp.float32) + bias
    t = jnp.dot(w, m_acc / _NTOT, preferred_element_type=jnp.float32)
    ey2 = jnp.sum(t * w, axis=1)[None, :]
    ey2 = ey2 + 2.0 * bias * mean - bias * bias
    var = ey2 - mean * mean
    sc = gamma * lax.rsqrt(var + _EPS)
    at = wt * sc
    c = sc * (bias - mean) + beta
    return at, c


def _passA_body(g_ref, w_ref, wt_ref, b_ref, ga_ref, be_ref,
                at_ref, c_ref, s_acc, m_acc):
    @pl.when(pl.program_id(0) == 0)
    def _():
        s_acc[...] = jnp.zeros_like(s_acc)
        m_acc[...] = jnp.zeros_like(m_acc)

    g = g_ref[...]
    s_acc[...] += jnp.sum(g, axis=0, keepdims=True)
    m_acc[...] += lax.dot_general(g, g, (((0,), (0,)), ((), ())),
                                  preferred_element_type=jnp.float32)

    @pl.when(pl.program_id(0) == _NBLK - 1)
    def _():
        at, c = _fold_stats(s_acc[...], m_acc[...], w_ref[...], wt_ref[...],
                            b_ref[...], ga_ref[...], be_ref[...])
        at_ref[...] = at
        c_ref[...] = c


def _passB_body(g_ref, a1t_ref, c1_ref, w_ref, wt_ref, b_ref, ga_ref, be_ref,
                at_ref, c_ref, s_acc, m_acc):
    @pl.when(pl.program_id(0) == 0)
    def _():
        s_acc[...] = jnp.zeros_like(s_acc)
        m_acc[...] = jnp.zeros_like(m_acc)

    g = g_ref[...]
    r = jnp.maximum(jnp.dot(g, a1t_ref[...],
                            preferred_element_type=jnp.float32)
                    + c1_ref[...], 0.0)
    s_acc[...] += jnp.sum(r, axis=0, keepdims=True)
    m_acc[...] += lax.dot_general(r, r, (((0,), (0,)), ((), ())),
                                  preferred_element_type=jnp.float32)

    @pl.when(pl.program_id(0) == _NBLK - 1)
    def _():
        at, c = _fold_stats(s_acc[...], m_acc[...], w_ref[...], wt_ref[...],
                            b_ref[...], ga_ref[...], be_ref[...])
        at_ref[...] = at
        c_ref[...] = c


def _passC_body(g_ref, a1t_ref, c1_ref, a2t_ref, c2_ref, o_ref):
    g = g_ref[...]
    r = jnp.maximum(jnp.dot(g, a1t_ref[...],
                            preferred_element_type=jnp.float32)
                    + c1_ref[...], 0.0)
    o_ref[...] = (jnp.dot(r, a2t_ref[...], preferred_element_type=jnp.float32)
                  + c2_ref[...])


_full_spec = pl.BlockSpec((128, 128), lambda i: (0, 0))
_vec_spec = pl.BlockSpec((1, 128), lambda i: (0, 0))
_row_spec = pl.BlockSpec((_ROWBLK, 128), lambda i: (i, 0))


def _passA(G, w, wt, bias, gamma, beta):
    return pl.pallas_call(
        _passA_body,
        grid=(_NBLK,),
        in_specs=[_row_spec, _full_spec, _full_spec, _vec_spec, _vec_spec,
                  _vec_spec],
        out_specs=[_full_spec, _vec_spec],
        out_shape=[jax.ShapeDtypeStruct((128, 128), jnp.float32),
                   jax.ShapeDtypeStruct((1, 128), jnp.float32)],
        scratch_shapes=[pltpu.VMEM((1, 128), jnp.float32),
                        pltpu.VMEM((128, 128), jnp.float32)],
    )(G, w, wt, bias, gamma, beta)


def _passB(G, a1t, c1, w, wt, bias, gamma, beta):
    return pl.pallas_call(
        _passB_body,
        grid=(_NBLK,),
        in_specs=[_row_spec, _full_spec, _vec_spec, _full_spec, _full_spec,
                  _vec_spec, _vec_spec, _vec_spec],
        out_specs=[_full_spec, _vec_spec],
        out_shape=[jax.ShapeDtypeStruct((128, 128), jnp.float32),
                   jax.ShapeDtypeStruct((1, 128), jnp.float32)],
        scratch_shapes=[pltpu.VMEM((1, 128), jnp.float32),
                        pltpu.VMEM((128, 128), jnp.float32)],
    )(G, a1t, c1, w, wt, bias, gamma, beta)


def _passC(G, a1t, c1, a2t, c2):
    return pl.pallas_call(
        _passC_body,
        grid=(_NBLK,),
        in_specs=[_row_spec, _full_spec, _vec_spec, _full_spec, _vec_spec],
        out_specs=_row_spec,
        out_shape=jax.ShapeDtypeStruct((_B * _N1 * _K, _C), jnp.float32),
    )(G, a1t, c1, a2t, c2)


def _bf16_round(x):
    u = lax.bitcast_convert_type(x, jnp.uint32)
    u = (u + jnp.uint32(0x7FFF) + ((u >> 16) & jnp.uint32(1))) \
        & jnp.uint32(0xFFFF0000)
    return lax.bitcast_convert_type(u, jnp.float32)


def kernel(xyz, p1, feat, W1, b1, g1, be1, W2, b2, g2, be2):
    xyz_b = _bf16_round(xyz)
    p1_b = _bf16_round(p1)

    def _perm(a):
        return a.reshape(_B, 64, 16, 16).transpose(0, 3, 2, 1).reshape(_B, _N2)

    xp = _perm(xyz_b[:, :, 0])
    yp = _perm(xyz_b[:, :, 1])
    zp = _perm(xyz_b[:, :, 2])
    sp = _perm(jnp.sum(xyz * xyz, axis=-1))
    qx = p1_b[:, :, 0]
    qy = p1_b[:, :, 1]
    qz = p1_b[:, :, 2]
    qs = jnp.sum(p1 * p1, axis=-1)

    gidx = _knn_call(xp, yp, zp, sp, qx, qy, qz, qs)
    featflat = feat.reshape(_B * _N2, _C)
    G = _gather_call(featflat, gidx)

    w1t = W1.T
    w2t = W2.T
    b1r, g1r, be1r = b1[None, :], g1[None, :], be1[None, :]
    b2r, g2r, be2r = b2[None, :], g2[None, :], be2[None, :]
    a1t, c1 = _passA(G, W1, w1t, b1r, g1r, be1r)
    a2t, c2 = _passB(G, a1t, c1, W2, w2t, b2r, g2r, be2r)
    out = _passC(G, a1t, c1, a2t, c2)
    return out.reshape(_B, _N1, _K, _C)


# speedup vs baseline: 14.2893x; 1.2079x over previous
import functools

import jax
import jax.numpy as jnp
from jax import lax
from jax.experimental import pallas as pl
from jax.experimental.pallas import tpu as pltpu
from jax.experimental.pallas import tpu_sc as plsc

_B = 2
_N1 = 1024
_N2 = 16384
_C = 128
_K = 50
_NW = 32
_RPW = (_B * _N1) // _NW
_SPB = _N1 // _RPW
_NG = _N2 // 16
_NS = _NG // 16
_RK = _RPW * _K
_INF = 3.0e38
_EPS = 1e-5
_NTOT = float(_B * _N1 * _K)
_ROWBLK = 2048
_NBLK = (_B * _N1 * _K) // _ROWBLK


def _full16(v, dtype=jnp.int32):
    return jnp.full((16,), v, dtype)


def _knn_body(xp, yp, zp, sp, qx, qy, qz, qs, gidx,
              xb, yb, zb, sb, qxb, qyb, qzb, qsb, db, cmb, smb, ixb):
    wid = lax.axis_index("s") * 2 + lax.axis_index("c")
    b = wid // _SPB
    qoff = (wid % _SPB) * _RPW
    pltpu.sync_copy(xp.at[b], xb)
    pltpu.sync_copy(yp.at[b], yb)
    pltpu.sync_copy(zp.at[b], zb)
    pltpu.sync_copy(sp.at[b], sb)
    pltpu.sync_copy(qx.at[b, pl.ds(qoff, _RPW)], qxb)
    pltpu.sync_copy(qy.at[b, pl.ds(qoff, _RPW)], qyb)
    pltpu.sync_copy(qz.at[b, pl.ds(qoff, _RPW)], qzb)
    pltpu.sync_copy(qs.at[b, pl.ds(qoff, _RPW)], qsb)
    iota16 = lax.iota(jnp.int32, 16)
    batch_base = b * _N2

    def qbody(q, _):
        qi = _full16(q)
        qxs = plsc.load_gather(qxb, [qi])
        qys = plsc.load_gather(qyb, [qi])
        qzs = plsc.load_gather(qzb, [qi])
        qss = plsc.load_gather(qsb, [qi])

        for u in range(4):
            def vbody(v2, smacc, u=u):
                base = (u + 4 * v2) * 16
                gm = jnp.full((16,), _INF)
                for m in range(16):
                    off = m * _NG + base
                    xv = xb[pl.ds(off, 16)]
                    yv = yb[pl.ds(off, 16)]
                    zv = zb[pl.ds(off, 16)]
                    sv = sb[pl.ds(off, 16)]
                    d = (qss + sv) - 2.0 * ((qxs * xv + qys * yv)
                                            + qzs * zv)
                    db[pl.ds(off, 16)] = d
                    gm = jnp.minimum(gm, d)
                cmb[pl.ds(base, 16)] = gm
                return jnp.minimum(smacc, gm)
            smacc = lax.fori_loop(0, 16, vbody, jnp.full((16,), _INF))
            smb[pl.ds(u * 16, 16)] = smacc

        rowbase = q * _K
        for vec_i in range(4):
            nr = 16 if vec_i < 3 else 2

            def rbody(r2, outvec):
                rm = jnp.full((16,), _INF)
                ra = jnp.zeros((16,), jnp.int32)
                for i in range(4):
                    smv = smb[pl.ds(i * 16, 16)]
                    msk = smv < rm
                    rm = jnp.where(msk, smv, rm)
                    ra = jnp.where(msk, jnp.full((16,), i, jnp.int32), ra)
                m = jnp.min(rm)
                key = jnp.where(rm == m, ra * 16 + iota16,
                                jnp.full((16,), 1024, jnp.int32))
                s_star = jnp.min(key)
                cmg = plsc.load_gather(cmb, [iota16 * _NS + s_star])
                m2_s = jnp.max(plsc.all_reduce_ffs(cmg == m))
                g_star = m2_s * _NS + s_star
                dg = plsc.load_gather(db, [iota16 * _NG + g_star])
                mm_s = jnp.max(plsc.all_reduce_ffs(dg == m))
                j_loc = mm_s * _NG + g_star
                j_cand = s_star * 256 + m2_s * 16 + mm_s
                outvec = jnp.where(iota16 == r2,
                                   _full16(j_cand + batch_base), outvec)
                lane0 = iota16 == 0
                plsc.store_scatter(db, [_full16(j_loc)],
                                   jnp.full((16,), _INF), mask=lane0)
                ncm = jnp.min(jnp.where(iota16 == mm_s, _INF, dg))
                plsc.store_scatter(cmb, [_full16(g_star)],
                                   jnp.full((16,), ncm), mask=lane0)
                nsm = jnp.min(jnp.where(iota16 == m2_s, ncm, cmg))
                plsc.store_scatter(smb, [_full16(s_star)],
                                   jnp.full((16,), nsm), mask=lane0)
                return outvec

            outvec = lax.fori_loop(0, nr, rbody, jnp.zeros((16,), jnp.int32))
            ixb[pl.ds(rowbase + vec_i * 16, 16)] = outvec
        return 0

    lax.fori_loop(0, _RPW, qbody, 0)
    pltpu.sync_copy(ixb.at[pl.ds(0, _RK)], gidx.at[pl.ds(wid * _RK, _RK)])


def _knn_call(xp, yp, zp, sp, qx, qy, qz, qs):
    return pl.kernel(
        _knn_body,
        out_type=jax.ShapeDtypeStruct((_B * _N1 * _K,), jnp.int32),
        mesh=plsc.VectorSubcoreMesh(core_axis_name="c", subcore_axis_name="s"),
        compiler_params=pltpu.CompilerParams(needs_layout_passes=False),
        scratch_types=[
            pltpu.VMEM((_N2,), jnp.float32),
            pltpu.VMEM((_N2,), jnp.float32),
            pltpu.VMEM((_N2,), jnp.float32),
            pltpu.VMEM((_N2,), jnp.float32),
            pltpu.VMEM((_RPW,), jnp.float32),
            pltpu.VMEM((_RPW,), jnp.float32),
            pltpu.VMEM((_RPW,), jnp.float32),
            pltpu.VMEM((_RPW,), jnp.float32),
            pltpu.VMEM((_N2,), jnp.float32),
            pltpu.VMEM((_NG,), jnp.float32),
            pltpu.VMEM((_NS,), jnp.float32),
            pltpu.VMEM((_RK + 16,), jnp.int32),
        ],
    )(xp, yp, zp, sp, qx, qy, qz, qs)


def _gather_body(feat, gidx, out, idxb, rowb0, rowb1, sem0, sem1):
    wid = lax.axis_index("s") * 2 + lax.axis_index("c")
    nchunks = _RK // 128
    pltpu.sync_copy(gidx.at[pl.ds(wid * _RK, _RK)], idxb)
    bufs = (rowb0, rowb1)
    sems = (sem0, sem1)
    pltpu.async_copy(feat.at[idxb.at[pl.ds(0, 128)]], rowb0, sem0)

    def cbody(c, _):
        for par in range(2):
            @pl.when((c % 2) == par)
            def _(par=par):
                nxt = feat.at[idxb.at[pl.ds((c + 1) * 128, 128)]]
                @pl.when(c + 1 < nchunks)
                def _():
                    pltpu.async_copy(nxt, bufs[1 - par], sems[1 - par])
                pltpu.make_async_copy(nxt, bufs[par], sems[par]).wait()
                pltpu.sync_copy(bufs[par],
                                out.at[pl.ds(wid * _RK + c * 128, 128)])
        return 0

    lax.fori_loop(0, nchunks, cbody, 0)


def _gather_call(featflat, gidx):
    return pl.kernel(
        _gather_body,
        out_type=jax.ShapeDtypeStruct((_B * _N1 * _K, _C), jnp.float32),
        mesh=plsc.VectorSubcoreMesh(core_axis_name="c", subcore_axis_name="s"),
        compiler_params=pltpu.CompilerParams(needs_layout_passes=False),
        scratch_types=[
            pltpu.VMEM((_RK,), jnp.int32),
            pltpu.VMEM((128, _C), jnp.float32),
            pltpu.VMEM((128, _C), jnp.float32),
            pltpu.SemaphoreType.DMA,
            pltpu.SemaphoreType.DMA,
        ],
    )(featflat, gidx)


def _fold_stats(s_acc, m_acc, w, wt, bias, gamma, beta):
    mu = s_acc / _NTOT
    mean = jnp.dot(mu, wt, preferred_element_type=jnp.float32) + bias
    t = jnp.dot(w, m_acc / _NTOT, preferred_element_type=jnp.float32)
    ey2 = jnp.sum(t * w, axis=1)[None, :]
    ey2 = ey2 + 2.0 * bias * mean - bias * bias
    var = ey2 - mean * mean
    sc = gamma * lax.rsqrt(var + _EPS)
    at = wt * sc
    c = sc * (bias - mean) + beta
    return at, c


def _passA_body(g_ref, w_ref, wt_ref, b_ref, ga_ref, be_ref,
                at_ref, c_ref, s_acc, m_acc):
    @pl.when(pl.program_id(0) == 0)
    def _():
        s_acc[...] = jnp.zeros_like(s_acc)
        m_acc[...] = jnp.zeros_like(m_acc)

    g = g_ref[...]
    s_acc[...] += jnp.sum(g, axis=0, keepdims=True)
    m_acc[...] += lax.dot_general(g, g, (((0,), (0,)), ((), ())),
                                  preferred_element_type=jnp.float32)

    @pl.when(pl.program_id(0) == _NBLK - 1)
    def _():
        at, c = _fold_stats(s_acc[...], m_acc[...], w_ref[...], wt_ref[...],
                            b_ref[...], ga_ref[...], be_ref[...])
        at_ref[...] = at
        c_ref[...] = c


def _passB_body(g_ref, a1t_ref, c1_ref, w_ref, wt_ref, b_ref, ga_ref, be_ref,
                at_ref, c_ref, s_acc, m_acc):
    @pl.when(pl.program_id(0) == 0)
    def _():
        s_acc[...] = jnp.zeros_like(s_acc)
        m_acc[...] = jnp.zeros_like(m_acc)

    g = g_ref[...]
    r = jnp.maximum(jnp.dot(g, a1t_ref[...],
                            preferred_element_type=jnp.float32)
                    + c1_ref[...], 0.0)
    s_acc[...] += jnp.sum(r, axis=0, keepdims=True)
    m_acc[...] += lax.dot_general(r, r, (((0,), (0,)), ((), ())),
                                  preferred_element_type=jnp.float32)

    @pl.when(pl.program_id(0) == _NBLK - 1)
    def _():
        at, c = _fold_stats(s_acc[...], m_acc[...], w_ref[...], wt_ref[...],
                            b_ref[...], ga_ref[...], be_ref[...])
        at_ref[...] = at
        c_ref[...] = c


def _passC_body(g_ref, a1t_ref, c1_ref, a2t_ref, c2_ref, o_ref):
    g = g_ref[...]
    r = jnp.maximum(jnp.dot(g, a1t_ref[...],
                            preferred_element_type=jnp.float32)
                    + c1_ref[...], 0.0)
    o_ref[...] = (jnp.dot(r, a2t_ref[...], preferred_element_type=jnp.float32)
                  + c2_ref[...])


_full_spec = pl.BlockSpec((128, 128), lambda i: (0, 0))
_vec_spec = pl.BlockSpec((1, 128), lambda i: (0, 0))
_row_spec = pl.BlockSpec((_ROWBLK, 128), lambda i: (i, 0))


def _passA(G, w, wt, bias, gamma, beta):
    return pl.pallas_call(
        _passA_body,
        grid=(_NBLK,),
        in_specs=[_row_spec, _full_spec, _full_spec, _vec_spec, _vec_spec,
                  _vec_spec],
        out_specs=[_full_spec, _vec_spec],
        out_shape=[jax.ShapeDtypeStruct((128, 128), jnp.float32),
                   jax.ShapeDtypeStruct((1, 128), jnp.float32)],
        scratch_shapes=[pltpu.VMEM((1, 128), jnp.float32),
                        pltpu.VMEM((128, 128), jnp.float32)],
    )(G, w, wt, bias, gamma, beta)


def _passB(G, a1t, c1, w, wt, bias, gamma, beta):
    return pl.pallas_call(
        _passB_body,
        grid=(_NBLK,),
        in_specs=[_row_spec, _full_spec, _vec_spec, _full_spec, _full_spec,
                  _vec_spec, _vec_spec, _vec_spec],
        out_specs=[_full_spec, _vec_spec],
        out_shape=[jax.ShapeDtypeStruct((128, 128), jnp.float32),
                   jax.ShapeDtypeStruct((1, 128), jnp.float32)],
        scratch_shapes=[pltpu.VMEM((1, 128), jnp.float32),
                        pltpu.VMEM((128, 128), jnp.float32)],
    )(G, a1t, c1, w, wt, bias, gamma, beta)


def _passC(G, a1t, c1, a2t, c2):
    return pl.pallas_call(
        _passC_body,
        grid=(_NBLK,),
        in_specs=[_row_spec, _full_spec, _vec_spec, _full_spec, _vec_spec],
        out_specs=_row_spec,
        out_shape=jax.ShapeDtypeStruct((_B * _N1 * _K, _C), jnp.float32),
    )(G, a1t, c1, a2t, c2)


def _bf16_round(x):
    u = lax.bitcast_convert_type(x, jnp.uint32)
    u = (u + jnp.uint32(0x7FFF) + ((u >> 16) & jnp.uint32(1))) \
        & jnp.uint32(0xFFFF0000)
    return lax.bitcast_convert_type(u, jnp.float32)


def kernel(xyz, p1, feat, W1, b1, g1, be1, W2, b2, g2, be2):
    xyz_b = _bf16_round(xyz)
    p1_b = _bf16_round(p1)

    def _perm(a):
        return a.reshape(_B, 64, 16, 16).transpose(0, 3, 2, 1).reshape(_B, _N2)

    xp = _perm(xyz_b[:, :, 0])
    yp = _perm(xyz_b[:, :, 1])
    zp = _perm(xyz_b[:, :, 2])
    sp = _perm(jnp.sum(xyz * xyz, axis=-1))
    qx = p1_b[:, :, 0]
    qy = p1_b[:, :, 1]
    qz = p1_b[:, :, 2]
    qs = jnp.sum(p1 * p1, axis=-1)

    gidx = _knn_call(xp, yp, zp, sp, qx, qy, qz, qs)
    featflat = feat.reshape(_B * _N2, _C)
    G = _gather_call(featflat, gidx)

    w1t = W1.T
    w2t = W2.T
    b1r, g1r, be1r = b1[None, :], g1[None, :], be1[None, :]
    b2r, g2r, be2r = b2[None, :], g2[None, :], be2[None, :]
    a1t, c1 = _passA(G, W1, w1t, b1r, g1r, be1r)
    a2t, c2 = _passB(G, a1t, c1, W2, w2t, b2r, g2r, be2r)
    out = _passC(G, a1t, c1, a2t, c2)
    return out.reshape(_B, _N1, _K, _C)


# speedup vs baseline: 14.3037x; 1.0010x over previous
import functools

import jax
import jax.numpy as jnp
from jax import lax
from jax.experimental import pallas as pl
from jax.experimental.pallas import tpu as pltpu
from jax.experimental.pallas import tpu_sc as plsc

_B = 2
_N1 = 1024
_N2 = 16384
_C = 128
_K = 50
_NW = 32
_RPW = (_B * _N1) // _NW
_SPB = _N1 // _RPW
_NG = _N2 // 16
_NS = _NG // 16
_RK = _RPW * _K
_INF = 3.0e38
_EPS = 1e-5
_NTOT = float(_B * _N1 * _K)
_ROWBLK = 2048
_NBLK = (_B * _N1 * _K) // _ROWBLK


def _full16(v, dtype=jnp.int32):
    return jnp.full((16,), v, dtype)


def _knn_body(xp, yp, zp, sp, qx, qy, qz, qs, gidx,
              xb, yb, zb, sb, qxb, qyb, qzb, qsb,
              db0, db1, cmb0, cmb1, smb0, smb1, ixb):
    wid = lax.axis_index("s") * 2 + lax.axis_index("c")
    b = wid // _SPB
    qoff = (wid % _SPB) * _RPW
    pltpu.sync_copy(xp.at[b], xb)
    pltpu.sync_copy(yp.at[b], yb)
    pltpu.sync_copy(zp.at[b], zb)
    pltpu.sync_copy(sp.at[b], sb)
    pltpu.sync_copy(qx.at[b, pl.ds(qoff, _RPW)], qxb)
    pltpu.sync_copy(qy.at[b, pl.ds(qoff, _RPW)], qyb)
    pltpu.sync_copy(qz.at[b, pl.ds(qoff, _RPW)], qzb)
    pltpu.sync_copy(qs.at[b, pl.ds(qoff, _RPW)], qsb)
    iota16 = lax.iota(jnp.int32, 16)
    batch_base = b * _N2

    def round_step(db_, cmb_, smb_, r2, outvec):
        rm = jnp.full((16,), _INF)
        ra = jnp.zeros((16,), jnp.int32)
        for i in range(4):
            smv = smb_[pl.ds(i * 16, 16)]
            msk = smv < rm
            rm = jnp.where(msk, smv, rm)
            ra = jnp.where(msk, jnp.full((16,), i, jnp.int32), ra)
        m = jnp.min(rm)
        key = jnp.where(rm == m, ra * 16 + iota16,
                        jnp.full((16,), 1024, jnp.int32))
        s_star = jnp.min(key)
        cmg = plsc.load_gather(cmb_, [iota16 * _NS + s_star])
        m2_s = jnp.max(plsc.all_reduce_ffs(cmg == m))
        g_star = m2_s * _NS + s_star
        dg = plsc.load_gather(db_, [iota16 * _NG + g_star])
        mm_s = jnp.max(plsc.all_reduce_ffs(dg == m))
        j_loc = mm_s * _NG + g_star
        j_cand = s_star * 256 + m2_s * 16 + mm_s
        outvec = jnp.where(iota16 == r2,
                           _full16(j_cand + batch_base), outvec)
        lane0 = iota16 == 0
        plsc.store_scatter(db_, [_full16(j_loc)],
                           jnp.full((16,), _INF), mask=lane0)
        ncm = jnp.min(jnp.where(iota16 == mm_s, _INF, dg))
        plsc.store_scatter(cmb_, [_full16(g_star)],
                           jnp.full((16,), ncm), mask=lane0)
        nsm = jnp.min(jnp.where(iota16 == m2_s, ncm, cmg))
        plsc.store_scatter(smb_, [_full16(s_star)],
                           jnp.full((16,), nsm), mask=lane0)
        return outvec

    def qbody(q, _):
        qi0 = _full16(q)
        qi1 = _full16(q + _RPW // 2)
        qxs0 = plsc.load_gather(qxb, [qi0])
        qys0 = plsc.load_gather(qyb, [qi0])
        qzs0 = plsc.load_gather(qzb, [qi0])
        qss0 = plsc.load_gather(qsb, [qi0])
        qxs1 = plsc.load_gather(qxb, [qi1])
        qys1 = plsc.load_gather(qyb, [qi1])
        qzs1 = plsc.load_gather(qzb, [qi1])
        qss1 = plsc.load_gather(qsb, [qi1])

        for u in range(4):
            def vbody(v2, accs, u=u):
                sma0, sma1 = accs
                base = (u + 4 * v2) * 16
                gm0 = jnp.full((16,), _INF)
                gm1 = jnp.full((16,), _INF)
                for m in range(16):
                    off = m * _NG + base
                    xv = xb[pl.ds(off, 16)]
                    yv = yb[pl.ds(off, 16)]
                    zv = zb[pl.ds(off, 16)]
                    sv = sb[pl.ds(off, 16)]
                    d0 = (qss0 + sv) - 2.0 * ((qxs0 * xv + qys0 * yv)
                                              + qzs0 * zv)
                    d1 = (qss1 + sv) - 2.0 * ((qxs1 * xv + qys1 * yv)
                                              + qzs1 * zv)
                    db0[pl.ds(off, 16)] = d0
                    db1[pl.ds(off, 16)] = d1
                    gm0 = jnp.minimum(gm0, d0)
                    gm1 = jnp.minimum(gm1, d1)
                cmb0[pl.ds(base, 16)] = gm0
                cmb1[pl.ds(base, 16)] = gm1
                return (jnp.minimum(sma0, gm0), jnp.minimum(sma1, gm1))
            sma0, sma1 = lax.fori_loop(
                0, 16, vbody,
                (jnp.full((16,), _INF), jnp.full((16,), _INF)))
            smb0[pl.ds(u * 16, 16)] = sma0
            smb1[pl.ds(u * 16, 16)] = sma1

        rb0 = q * _K
        rb1 = (q + _RPW // 2) * _K
        for vec_i in range(4):
            nr = 16 if vec_i < 3 else 2

            def rbody(r2, ovs):
                ov0, ov1 = ovs
                ov0 = round_step(db0, cmb0, smb0, r2, ov0)
                ov1 = round_step(db1, cmb1, smb1, r2, ov1)
                return (ov0, ov1)

            ov0, ov1 = lax.fori_loop(
                0, nr, rbody,
                (jnp.zeros((16,), jnp.int32), jnp.zeros((16,), jnp.int32)))
            ixb[pl.ds(rb0 + vec_i * 16, 16)] = ov0
            ixb[pl.ds(rb1 + vec_i * 16, 16)] = ov1
        return 0

    lax.fori_loop(0, _RPW // 2, qbody, 0)
    pltpu.sync_copy(ixb.at[pl.ds(0, _RK)], gidx.at[pl.ds(wid * _RK, _RK)])


def _knn_call(xp, yp, zp, sp, qx, qy, qz, qs):
    return pl.kernel(
        _knn_body,
        out_type=jax.ShapeDtypeStruct((_B * _N1 * _K,), jnp.int32),
        mesh=plsc.VectorSubcoreMesh(core_axis_name="c", subcore_axis_name="s"),
        compiler_params=pltpu.CompilerParams(needs_layout_passes=False),
        scratch_types=[
            pltpu.VMEM((_N2,), jnp.float32),
            pltpu.VMEM((_N2,), jnp.float32),
            pltpu.VMEM((_N2,), jnp.float32),
            pltpu.VMEM((_N2,), jnp.float32),
            pltpu.VMEM((_RPW,), jnp.float32),
            pltpu.VMEM((_RPW,), jnp.float32),
            pltpu.VMEM((_RPW,), jnp.float32),
            pltpu.VMEM((_RPW,), jnp.float32),
            pltpu.VMEM((_N2,), jnp.float32),
            pltpu.VMEM((_N2,), jnp.float32),
            pltpu.VMEM((_NG,), jnp.float32),
            pltpu.VMEM((_NG,), jnp.float32),
            pltpu.VMEM((_NS,), jnp.float32),
            pltpu.VMEM((_NS,), jnp.float32),
            pltpu.VMEM((_RK + 16,), jnp.int32),
        ],
    )(xp, yp, zp, sp, qx, qy, qz, qs)


def _gather_body(feat, gidx, out, idxb, rowb0, rowb1, sem0, sem1):
    wid = lax.axis_index("s") * 2 + lax.axis_index("c")
    nchunks = _RK // 128
    pltpu.sync_copy(gidx.at[pl.ds(wid * _RK, _RK)], idxb)
    bufs = (rowb0, rowb1)
    sems = (sem0, sem1)
    pltpu.async_copy(feat.at[idxb.at[pl.ds(0, 128)]], rowb0, sem0)

    def cbody(c, _):
        for par in range(2):
            @pl.when((c % 2) == par)
            def _(par=par):
                nxt = feat.at[idxb.at[pl.ds((c + 1) * 128, 128)]]
                @pl.when(c + 1 < nchunks)
                def _():
                    pltpu.async_copy(nxt, bufs[1 - par], sems[1 - par])
                pltpu.make_async_copy(nxt, bufs[par], sems[par]).wait()
                pltpu.sync_copy(bufs[par],
                                out.at[pl.ds(wid * _RK + c * 128, 128)])
        return 0

    lax.fori_loop(0, nchunks, cbody, 0)


def _gather_call(featflat, gidx):
    return pl.kernel(
        _gather_body,
        out_type=jax.ShapeDtypeStruct((_B * _N1 * _K, _C), jnp.float32),
        mesh=plsc.VectorSubcoreMesh(core_axis_name="c", subcore_axis_name="s"),
        compiler_params=pltpu.CompilerParams(needs_layout_passes=False),
        scratch_types=[
            pltpu.VMEM((_RK,), jnp.int32),
            pltpu.VMEM((128, _C), jnp.float32),
            pltpu.VMEM((128, _C), jnp.float32),
            pltpu.SemaphoreType.DMA,
            pltpu.SemaphoreType.DMA,
        ],
    )(featflat, gidx)


def _fold_stats(s_acc, m_acc, w, wt, bias, gamma, beta):
    mu = s_acc / _NTOT
    mean = jnp.dot(mu, wt, preferred_element_type=jnp.float32) + bias
    t = jnp.dot(w, m_acc / _NTOT, preferred_element_type=jnp.float32)
    ey2 = jnp.sum(t * w, axis=1)[None, :]
    ey2 = ey2 + 2.0 * bias * mean - bias * bias
    var = ey2 - mean * mean
    sc = gamma * lax.rsqrt(var + _EPS)
    at = wt * sc
    c = sc * (bias - mean) + beta
    return at, c


def _passA_body(g_ref, w_ref, wt_ref, b_ref, ga_ref, be_ref,
                at_ref, c_ref, s_acc, m_acc):
    @pl.when(pl.program_id(0) == 0)
    def _():
        s_acc[...] = jnp.zeros_like(s_acc)
        m_acc[...] = jnp.zeros_like(m_acc)

    g = g_ref[...]
    s_acc[...] += jnp.sum(g, axis=0, keepdims=True)
    m_acc[...] += lax.dot_general(g, g, (((0,), (0,)), ((), ())),
                                  preferred_element_type=jnp.float32)

    @pl.when(pl.program_id(0) == _NBLK - 1)
    def _():
        at, c = _fold_stats(s_acc[...], m_acc[...], w_ref[...], wt_ref[...],
                            b_ref[...], ga_ref[...], be_ref[...])
        at_ref[...] = at
        c_ref[...] = c


def _passB_body(g_ref, a1t_ref, c1_ref, w_ref, wt_ref, b_ref, ga_ref, be_ref,
                at_ref, c_ref, s_acc, m_acc):
    @pl.when(pl.program_id(0) == 0)
    def _():
        s_acc[...] = jnp.zeros_like(s_acc)
        m_acc[...] = jnp.zeros_like(m_acc)

    g = g_ref[...]
    r = jnp.maximum(jnp.dot(g, a1t_ref[...],
                            preferred_element_type=jnp.float32)
                    + c1_ref[...], 0.0)
    s_acc[...] += jnp.sum(r, axis=0, keepdims=True)
    m_acc[...] += lax.dot_general(r, r, (((0,), (0,)), ((), ())),
                                  preferred_element_type=jnp.float32)

    @pl.when(pl.program_id(0) == _NBLK - 1)
    def _():
        at, c = _fold_stats(s_acc[...], m_acc[...], w_ref[...], wt_ref[...],
                            b_ref[...], ga_ref[...], be_ref[...])
        at_ref[...] = at
        c_ref[...] = c


def _passC_body(g_ref, a1t_ref, c1_ref, a2t_ref, c2_ref, o_ref):
    g = g_ref[...]
    r = jnp.maximum(jnp.dot(g, a1t_ref[...],
                            preferred_element_type=jnp.float32)
                    + c1_ref[...], 0.0)
    o_ref[...] = (jnp.dot(r, a2t_ref[...], preferred_element_type=jnp.float32)
                  + c2_ref[...])


_full_spec = pl.BlockSpec((128, 128), lambda i: (0, 0))
_vec_spec = pl.BlockSpec((1, 128), lambda i: (0, 0))
_row_spec = pl.BlockSpec((_ROWBLK, 128), lambda i: (i, 0))


def _passA(G, w, wt, bias, gamma, beta):
    return pl.pallas_call(
        _passA_body,
        grid=(_NBLK,),
        in_specs=[_row_spec, _full_spec, _full_spec, _vec_spec, _vec_spec,
                  _vec_spec],
        out_specs=[_full_spec, _vec_spec],
        out_shape=[jax.ShapeDtypeStruct((128, 128), jnp.float32),
                   jax.ShapeDtypeStruct((1, 128), jnp.float32)],
        scratch_shapes=[pltpu.VMEM((1, 128), jnp.float32),
                        pltpu.VMEM((128, 128), jnp.float32)],
    )(G, w, wt, bias, gamma, beta)


def _passB(G, a1t, c1, w, wt, bias, gamma, beta):
    return pl.pallas_call(
        _passB_body,
        grid=(_NBLK,),
        in_specs=[_row_spec, _full_spec, _vec_spec, _full_spec, _full_spec,
                  _vec_spec, _vec_spec, _vec_spec],
        out_specs=[_full_spec, _vec_spec],
        out_shape=[jax.ShapeDtypeStruct((128, 128), jnp.float32),
                   jax.ShapeDtypeStruct((1, 128), jnp.float32)],
        scratch_shapes=[pltpu.VMEM((1, 128), jnp.float32),
                        pltpu.VMEM((128, 128), jnp.float32)],
    )(G, a1t, c1, w, wt, bias, gamma, beta)


def _passC(G, a1t, c1, a2t, c2):
    return pl.pallas_call(
        _passC_body,
        grid=(_NBLK,),
        in_specs=[_row_spec, _full_spec, _vec_spec, _full_spec, _vec_spec],
        out_specs=_row_spec,
        out_shape=jax.ShapeDtypeStruct((_B * _N1 * _K, _C), jnp.float32),
    )(G, a1t, c1, a2t, c2)


def _bf16_round(x):
    u = lax.bitcast_convert_type(x, jnp.uint32)
    u = (u + jnp.uint32(0x7FFF) + ((u >> 16) & jnp.uint32(1))) \
        & jnp.uint32(0xFFFF0000)
    return lax.bitcast_convert_type(u, jnp.float32)


def kernel(xyz, p1, feat, W1, b1, g1, be1, W2, b2, g2, be2):
    xyz_b = _bf16_round(xyz)
    p1_b = _bf16_round(p1)

    def _perm(a):
        return a.reshape(_B, 64, 16, 16).transpose(0, 3, 2, 1).reshape(_B, _N2)

    xp = _perm(xyz_b[:, :, 0])
    yp = _perm(xyz_b[:, :, 1])
    zp = _perm(xyz_b[:, :, 2])
    sp = _perm(jnp.sum(xyz * xyz, axis=-1))
    qx = p1_b[:, :, 0]
    qy = p1_b[:, :, 1]
    qz = p1_b[:, :, 2]
    qs = jnp.sum(p1 * p1, axis=-1)

    gidx = _knn_call(xp, yp, zp, sp, qx, qy, qz, qs)
    featflat = feat.reshape(_B * _N2, _C)
    G = _gather_call(featflat, gidx)

    w1t = W1.T
    w2t = W2.T
    b1r, g1r, be1r = b1[None, :], g1[None, :], be1[None, :]
    b2r, g2r, be2r = b2[None, :], g2[None, :], be2[None, :]
    a1t, c1 = _passA(G, W1, w1t, b1r, g1r, be1r)
    a2t, c2 = _passB(G, a1t, c1, W2, w2t, b2r, g2r, be2r)
    out = _passC(G, a1t, c1, a2t, c2)
    return out.reshape(_B, _N1, _K, _C)


# speedup vs baseline: 14.6556x; 1.0246x over previous
import functools

import jax
import jax.numpy as jnp
from jax import lax
from jax.experimental import pallas as pl
from jax.experimental.pallas import tpu as pltpu
from jax.experimental.pallas import tpu_sc as plsc

_B = 2
_N1 = 1024
_N2 = 16384
_C = 128
_K = 50
_NW = 32
_RPW = (_B * _N1) // _NW
_SPB = _N1 // _RPW
_NG = _N2 // 16
_NS = _NG // 16
_RK = _RPW * _K
_INF = 3.0e38
_EPS = 1e-5
_NTOT = float(_B * _N1 * _K)
_ROWBLK = 2048
_NBLK = (_B * _N1 * _K) // _ROWBLK


def _full16(v, dtype=jnp.int32):
    return jnp.full((16,), v, dtype)


def _knn_body(xp, yp, zp, sp, qx, qy, qz, qs, gidx,
              xb, yb, zb, sb, qxb, qyb, qzb, qsb,
              db0, db1, cmb0, cmb1, smb0, smb1, ixb):
    wid = lax.axis_index("s") * 2 + lax.axis_index("c")
    b = wid // _SPB
    qoff = (wid % _SPB) * _RPW
    pltpu.sync_copy(xp.at[b], xb)
    pltpu.sync_copy(yp.at[b], yb)
    pltpu.sync_copy(zp.at[b], zb)
    pltpu.sync_copy(sp.at[b], sb)
    pltpu.sync_copy(qx.at[b, pl.ds(qoff, _RPW)], qxb)
    pltpu.sync_copy(qy.at[b, pl.ds(qoff, _RPW)], qyb)
    pltpu.sync_copy(qz.at[b, pl.ds(qoff, _RPW)], qzb)
    pltpu.sync_copy(qs.at[b, pl.ds(qoff, _RPW)], qsb)
    iota16 = lax.iota(jnp.int32, 16)
    batch_base = b * _N2

    def round_step(db_, cmb_, smb_, r2, outvec):
        rm = jnp.full((16,), _INF)
        ra = jnp.zeros((16,), jnp.int32)
        for i in range(4):
            smv = smb_[pl.ds(i * 16, 16)]
            msk = smv < rm
            rm = jnp.where(msk, smv, rm)
            ra = jnp.where(msk, jnp.full((16,), i, jnp.int32), ra)
        m = jnp.min(rm)
        key = jnp.where(rm == m, ra * 16 + iota16,
                        jnp.full((16,), 1024, jnp.int32))
        s_star = jnp.min(key)
        cmg = plsc.load_gather(cmb_, [iota16 * _NS + s_star])
        m2_s = jnp.max(plsc.all_reduce_ffs(cmg == m))
        g_star = m2_s * _NS + s_star
        dg = plsc.load_gather(db_, [iota16 * _NG + g_star])
        mm_s = jnp.max(plsc.all_reduce_ffs(dg == m))
        j_loc = mm_s * _NG + g_star
        j_cand = s_star * 256 + m2_s * 16 + mm_s
        outvec = jnp.where(iota16 == r2,
                           _full16(j_cand + batch_base), outvec)
        lane0 = iota16 == 0
        plsc.store_scatter(db_, [_full16(j_loc)],
                           jnp.full((16,), _INF), mask=lane0)
        ncm = jnp.min(jnp.where(iota16 == mm_s, _INF, dg))
        plsc.store_scatter(cmb_, [_full16(g_star)],
                           jnp.full((16,), ncm), mask=lane0)
        nsm = jnp.min(jnp.where(iota16 == m2_s, ncm, cmg))
        plsc.store_scatter(smb_, [_full16(s_star)],
                           jnp.full((16,), nsm), mask=lane0)
        return outvec

    def qbody(q, _):
        qi0 = _full16(q)
        qi1 = _full16(q + _RPW // 2)
        qxs0 = plsc.load_gather(qxb, [qi0])
        qys0 = plsc.load_gather(qyb, [qi0])
        qzs0 = plsc.load_gather(qzb, [qi0])
        qss0 = plsc.load_gather(qsb, [qi0])
        qxs1 = plsc.load_gather(qxb, [qi1])
        qys1 = plsc.load_gather(qyb, [qi1])
        qzs1 = plsc.load_gather(qzb, [qi1])
        qss1 = plsc.load_gather(qsb, [qi1])

        for u in range(4):
            def vbody(v2, accs, u=u):
                sma0, sma1 = accs
                base = (u + 4 * v2) * 16
                gm0 = jnp.full((16,), _INF)
                gm1 = jnp.full((16,), _INF)
                for m in range(16):
                    off = m * _NG + base
                    xv = xb[pl.ds(off, 16)]
                    yv = yb[pl.ds(off, 16)]
                    zv = zb[pl.ds(off, 16)]
                    sv = sb[pl.ds(off, 16)]
                    d0 = (qss0 + sv) - 2.0 * ((qxs0 * xv + qys0 * yv)
                                              + qzs0 * zv)
                    d1 = (qss1 + sv) - 2.0 * ((qxs1 * xv + qys1 * yv)
                                              + qzs1 * zv)
                    db0[pl.ds(off, 16)] = d0
                    db1[pl.ds(off, 16)] = d1
                    gm0 = jnp.minimum(gm0, d0)
                    gm1 = jnp.minimum(gm1, d1)
                cmb0[pl.ds(base, 16)] = gm0
                cmb1[pl.ds(base, 16)] = gm1
                return (jnp.minimum(sma0, gm0), jnp.minimum(sma1, gm1))
            sma0, sma1 = lax.fori_loop(
                0, 16, vbody,
                (jnp.full((16,), _INF), jnp.full((16,), _INF)))
            smb0[pl.ds(u * 16, 16)] = sma0
            smb1[pl.ds(u * 16, 16)] = sma1

        rb0 = q * _K
        rb1 = (q + _RPW // 2) * _K
        for vec_i in range(4):
            nr = 16 if vec_i < 3 else 2

            def rbody(r2, ovs):
                ov0, ov1 = ovs
                ov0 = round_step(db0, cmb0, smb0, r2, ov0)
                ov1 = round_step(db1, cmb1, smb1, r2, ov1)
                return (ov0, ov1)

            ov0, ov1 = lax.fori_loop(
                0, nr, rbody,
                (jnp.zeros((16,), jnp.int32), jnp.zeros((16,), jnp.int32)))
            if vec_i < 3:
                ixb[pl.ds(rb0 + vec_i * 16, 16)] = ov0
                ixb[pl.ds(rb1 + vec_i * 16, 16)] = ov1
            else:
                lt2 = iota16 < 2
                plsc.store_compressed(ixb.at[pl.ds(rb0 + 48, 16)], ov0, mask=lt2)
                plsc.store_compressed(ixb.at[pl.ds(rb1 + 48, 16)], ov1, mask=lt2)
        return 0

    lax.fori_loop(0, _RPW // 2, qbody, 0)
    pltpu.sync_copy(ixb.at[pl.ds(0, _RK)], gidx.at[pl.ds(wid * _RK, _RK)])


def _knn_call(xp, yp, zp, sp, qx, qy, qz, qs):
    return pl.kernel(
        _knn_body,
        out_type=jax.ShapeDtypeStruct((_B * _N1 * _K,), jnp.int32),
        mesh=plsc.VectorSubcoreMesh(core_axis_name="c", subcore_axis_name="s"),
        compiler_params=pltpu.CompilerParams(needs_layout_passes=False),
        scratch_types=[
            pltpu.VMEM((_N2,), jnp.float32),
            pltpu.VMEM((_N2,), jnp.float32),
            pltpu.VMEM((_N2,), jnp.float32),
            pltpu.VMEM((_N2,), jnp.float32),
            pltpu.VMEM((_RPW,), jnp.float32),
            pltpu.VMEM((_RPW,), jnp.float32),
            pltpu.VMEM((_RPW,), jnp.float32),
            pltpu.VMEM((_RPW,), jnp.float32),
            pltpu.VMEM((_N2,), jnp.float32),
            pltpu.VMEM((_N2,), jnp.float32),
            pltpu.VMEM((_NG,), jnp.float32),
            pltpu.VMEM((_NG,), jnp.float32),
            pltpu.VMEM((_NS,), jnp.float32),
            pltpu.VMEM((_NS,), jnp.float32),
            pltpu.VMEM((_RK + 16,), jnp.int32),
        ],
    )(xp, yp, zp, sp, qx, qy, qz, qs)


def _gather_body(feat, gidx, out, idxb, rowb0, rowb1, sem0, sem1):
    wid = lax.axis_index("s") * 2 + lax.axis_index("c")
    nchunks = _RK // 128
    pltpu.sync_copy(gidx.at[pl.ds(wid * _RK, _RK)], idxb)
    bufs = (rowb0, rowb1)
    sems = (sem0, sem1)
    pltpu.async_copy(feat.at[idxb.at[pl.ds(0, 128)]], rowb0, sem0)

    def cbody(c, _):
        for par in range(2):
            @pl.when((c % 2) == par)
            def _(par=par):
                nxt = feat.at[idxb.at[pl.ds((c + 1) * 128, 128)]]
                @pl.when(c + 1 < nchunks)
                def _():
                    pltpu.async_copy(nxt, bufs[1 - par], sems[1 - par])
                pltpu.make_async_copy(nxt, bufs[par], sems[par]).wait()
                pltpu.sync_copy(bufs[par],
                                out.at[pl.ds(wid * _RK + c * 128, 128)])
        return 0

    lax.fori_loop(0, nchunks, cbody, 0)


def _gather_call(featflat, gidx):
    return pl.kernel(
        _gather_body,
        out_type=jax.ShapeDtypeStruct((_B * _N1 * _K, _C), jnp.float32),
        mesh=plsc.VectorSubcoreMesh(core_axis_name="c", subcore_axis_name="s"),
        compiler_params=pltpu.CompilerParams(needs_layout_passes=False),
        scratch_types=[
            pltpu.VMEM((_RK,), jnp.int32),
            pltpu.VMEM((128, _C), jnp.float32),
            pltpu.VMEM((128, _C), jnp.float32),
            pltpu.SemaphoreType.DMA,
            pltpu.SemaphoreType.DMA,
        ],
    )(featflat, gidx)


def _fold_stats(s_acc, m_acc, w, wt, bias, gamma, beta):
    mu = s_acc / _NTOT
    mean = jnp.dot(mu, wt, preferred_element_type=jnp.float32) + bias
    t = jnp.dot(w, m_acc / _NTOT, preferred_element_type=jnp.float32)
    ey2 = jnp.sum(t * w, axis=1)[None, :]
    ey2 = ey2 + 2.0 * bias * mean - bias * bias
    var = ey2 - mean * mean
    sc = gamma * lax.rsqrt(var + _EPS)
    at = wt * sc
    c = sc * (bias - mean) + beta
    return at, c


def _passA_body(g_ref, w_ref, wt_ref, b_ref, ga_ref, be_ref,
                at_ref, c_ref, s_acc, m_acc):
    @pl.when(pl.program_id(0) == 0)
    def _():
        s_acc[...] = jnp.zeros_like(s_acc)
        m_acc[...] = jnp.zeros_like(m_acc)

    g = g_ref[...]
    s_acc[...] += jnp.sum(g, axis=0, keepdims=True)
    m_acc[...] += lax.dot_general(g, g, (((0,), (0,)), ((), ())),
                                  preferred_element_type=jnp.float32)

    @pl.when(pl.program_id(0) == _NBLK - 1)
    def _():
        at, c = _fold_stats(s_acc[...], m_acc[...], w_ref[...], wt_ref[...],
                            b_ref[...], ga_ref[...], be_ref[...])
        at_ref[...] = at
        c_ref[...] = c


def _passB_body(g_ref, a1t_ref, c1_ref, w_ref, wt_ref, b_ref, ga_ref, be_ref,
                at_ref, c_ref, s_acc, m_acc):
    @pl.when(pl.program_id(0) == 0)
    def _():
        s_acc[...] = jnp.zeros_like(s_acc)
        m_acc[...] = jnp.zeros_like(m_acc)

    g = g_ref[...]
    r = jnp.maximum(jnp.dot(g, a1t_ref[...],
                            preferred_element_type=jnp.float32)
                    + c1_ref[...], 0.0)
    s_acc[...] += jnp.sum(r, axis=0, keepdims=True)
    m_acc[...] += lax.dot_general(r, r, (((0,), (0,)), ((), ())),
                                  preferred_element_type=jnp.float32)

    @pl.when(pl.program_id(0) == _NBLK - 1)
    def _():
        at, c = _fold_stats(s_acc[...], m_acc[...], w_ref[...], wt_ref[...],
                            b_ref[...], ga_ref[...], be_ref[...])
        at_ref[...] = at
        c_ref[...] = c


def _passC_body(g_ref, a1t_ref, c1_ref, a2t_ref, c2_ref, o_ref):
    g = g_ref[...]
    r = jnp.maximum(jnp.dot(g, a1t_ref[...],
                            preferred_element_type=jnp.float32)
                    + c1_ref[...], 0.0)
    o_ref[...] = (jnp.dot(r, a2t_ref[...], preferred_element_type=jnp.float32)
                  + c2_ref[...])


_full_spec = pl.BlockSpec((128, 128), lambda i: (0, 0))
_vec_spec = pl.BlockSpec((1, 128), lambda i: (0, 0))
_row_spec = pl.BlockSpec((_ROWBLK, 128), lambda i: (i, 0))


def _passA(G, w, wt, bias, gamma, beta):
    return pl.pallas_call(
        _passA_body,
        grid=(_NBLK,),
        in_specs=[_row_spec, _full_spec, _full_spec, _vec_spec, _vec_spec,
                  _vec_spec],
        out_specs=[_full_spec, _vec_spec],
        out_shape=[jax.ShapeDtypeStruct((128, 128), jnp.float32),
                   jax.ShapeDtypeStruct((1, 128), jnp.float32)],
        scratch_shapes=[pltpu.VMEM((1, 128), jnp.float32),
                        pltpu.VMEM((128, 128), jnp.float32)],
    )(G, w, wt, bias, gamma, beta)


def _passB(G, a1t, c1, w, wt, bias, gamma, beta):
    return pl.pallas_call(
        _passB_body,
        grid=(_NBLK,),
        in_specs=[_row_spec, _full_spec, _vec_spec, _full_spec, _full_spec,
                  _vec_spec, _vec_spec, _vec_spec],
        out_specs=[_full_spec, _vec_spec],
        out_shape=[jax.ShapeDtypeStruct((128, 128), jnp.float32),
                   jax.ShapeDtypeStruct((1, 128), jnp.float32)],
        scratch_shapes=[pltpu.VMEM((1, 128), jnp.float32),
                        pltpu.VMEM((128, 128), jnp.float32)],
    )(G, a1t, c1, w, wt, bias, gamma, beta)


def _passC(G, a1t, c1, a2t, c2):
    return pl.pallas_call(
        _passC_body,
        grid=(_NBLK,),
        in_specs=[_row_spec, _full_spec, _vec_spec, _full_spec, _vec_spec],
        out_specs=_row_spec,
        out_shape=jax.ShapeDtypeStruct((_B * _N1 * _K, _C), jnp.float32),
    )(G, a1t, c1, a2t, c2)


def _bf16_round(x):
    u = lax.bitcast_convert_type(x, jnp.uint32)
    u = (u + jnp.uint32(0x7FFF) + ((u >> 16) & jnp.uint32(1))) \
        & jnp.uint32(0xFFFF0000)
    return lax.bitcast_convert_type(u, jnp.float32)


def kernel(xyz, p1, feat, W1, b1, g1, be1, W2, b2, g2, be2):
    xyz_b = _bf16_round(xyz)
    p1_b = _bf16_round(p1)

    def _perm(a):
        return a.reshape(_B, 64, 16, 16).transpose(0, 3, 2, 1).reshape(_B, _N2)

    xp = _perm(xyz_b[:, :, 0])
    yp = _perm(xyz_b[:, :, 1])
    zp = _perm(xyz_b[:, :, 2])
    sp = _perm(jnp.sum(xyz * xyz, axis=-1))
    qx = p1_b[:, :, 0]
    qy = p1_b[:, :, 1]
    qz = p1_b[:, :, 2]
    qs = jnp.sum(p1 * p1, axis=-1)

    gidx = _knn_call(xp, yp, zp, sp, qx, qy, qz, qs)
    featflat = feat.reshape(_B * _N2, _C)
    G = _gather_call(featflat, gidx)

    w1t = W1.T
    w2t = W2.T
    b1r, g1r, be1r = b1[None, :], g1[None, :], be1[None, :]
    b2r, g2r, be2r = b2[None, :], g2[None, :], be2[None, :]
    a1t, c1 = _passA(G, W1, w1t, b1r, g1r, be1r)
    a2t, c2 = _passB(G, a1t, c1, W2, w2t, b2r, g2r, be2r)
    out = _passC(G, a1t, c1, a2t, c2)
    return out.reshape(_B, _N1, _K, _C)


# speedup vs baseline: 17.5918x; 1.2003x over previous
import functools

import jax
import jax.numpy as jnp
from jax import lax
from jax.experimental import pallas as pl
from jax.experimental.pallas import tpu as pltpu
from jax.experimental.pallas import tpu_sc as plsc

_B = 2
_N1 = 1024
_N2 = 16384
_C = 128
_K = 50
_NW = 32
_RPW = (_B * _N1) // _NW
_SPB = _N1 // _RPW
_NG = _N2 // 16
_NS = _NG // 16
_RK = _RPW * _K
_INF = 3.0e38
_EPS = 1e-5
_NTOT = float(_B * _N1 * _K)
_ROWBLK = 2048
_NBLK = (_B * _N1 * _K) // _ROWBLK


def _full16(v, dtype=jnp.int32):
    return jnp.full((16,), v, dtype)


def _knn_body(xp, yp, zp, sp, qx, qy, qz, qs, gidx,
              xb, yb, zb, sb, qxb, qyb, qzb, qsb, db, cmb, smb, ixb):
    wid = lax.axis_index("s") * 2 + lax.axis_index("c")
    b = wid // _SPB
    qoff = (wid % _SPB) * _RPW
    pltpu.sync_copy(xp.at[b], xb)
    pltpu.sync_copy(yp.at[b], yb)
    pltpu.sync_copy(zp.at[b], zb)
    pltpu.sync_copy(sp.at[b], sb)
    pltpu.sync_copy(qx.at[b, pl.ds(qoff, _RPW)], qxb)
    pltpu.sync_copy(qy.at[b, pl.ds(qoff, _RPW)], qyb)
    pltpu.sync_copy(qz.at[b, pl.ds(qoff, _RPW)], qzb)
    pltpu.sync_copy(qs.at[b, pl.ds(qoff, _RPW)], qsb)
    iota16 = lax.iota(jnp.int32, 16)
    batch_base = b * _N2

    def qbody(q, _):
        qi = _full16(q)
        qxs = plsc.load_gather(qxb, [qi])
        qys = plsc.load_gather(qyb, [qi])
        qzs = plsc.load_gather(qzb, [qi])
        qss = plsc.load_gather(qsb, [qi])

        for u in range(4):
            def vbody(v2, smacc, u=u):
                base = (u + 4 * v2) * 16
                gm = jnp.full((16,), _INF)
                for m in range(16):
                    off = m * _NG + base
                    xv = xb[pl.ds(off, 16)]
                    yv = yb[pl.ds(off, 16)]
                    zv = zb[pl.ds(off, 16)]
                    sv = sb[pl.ds(off, 16)]
                    d = (qss + sv) - 2.0 * ((qxs * xv + qys * yv)
                                            + qzs * zv)
                    db[pl.ds(off, 16)] = d
                    gm = jnp.minimum(gm, d)
                cmb[pl.ds(base, 16)] = gm
                return jnp.minimum(smacc, gm)
            smacc = lax.fori_loop(0, 16, vbody, jnp.full((16,), _INF))
            smb[pl.ds(u * 16, 16)] = smacc

        rowbase = q * _K
        for vec_i in range(4):
            nr = 16 if vec_i < 3 else 2

            def rbody(r2, outvec):
                rm = jnp.full((16,), _INF)
                ra = jnp.zeros((16,), jnp.int32)
                for i in range(4):
                    smv = smb[pl.ds(i * 16, 16)]
                    msk = smv < rm
                    rm = jnp.where(msk, smv, rm)
                    ra = jnp.where(msk, jnp.full((16,), i, jnp.int32), ra)
                m = jnp.min(rm)
                key = jnp.where(rm == m, ra * 16 + iota16,
                                jnp.full((16,), 1024, jnp.int32))
                s_star = jnp.min(key)
                cmg = plsc.load_gather(cmb, [iota16 * _NS + s_star])
                m2v = plsc.all_reduce_ffs(cmg == m)
                g_star = m2v * _NS + s_star
                dg = plsc.load_gather(db, [iota16 * _NG + g_star])
                mmv = plsc.all_reduce_ffs(dg == m)
                j_loc = mmv * _NG + g_star
                j_cand = s_star * 256 + m2v * 16 + mmv
                outvec = jnp.where(iota16 == r2, j_cand + batch_base, outvec)
                lane0 = iota16 == 0
                plsc.store_scatter(db, [j_loc],
                                   jnp.full((16,), _INF), mask=lane0)
                ncm = jnp.min(jnp.where(iota16 == mmv, _INF, dg))
                plsc.store_scatter(cmb, [g_star],
                                   jnp.full((16,), ncm), mask=lane0)
                nsm = jnp.min(jnp.where(iota16 == m2v, ncm, cmg))
                plsc.store_scatter(smb, [jnp.full((16,), s_star, jnp.int32)],
                                   jnp.full((16,), nsm), mask=lane0)
                return outvec

            outvec = lax.fori_loop(0, nr, rbody, jnp.zeros((16,), jnp.int32))
            if vec_i < 3:
                ixb[pl.ds(rowbase + vec_i * 16, 16)] = outvec
            else:
                plsc.store_compressed(ixb.at[pl.ds(rowbase + 48, 16)],
                                      outvec, mask=iota16 < 2)
        return 0

    lax.fori_loop(0, _RPW, qbody, 0)
    pltpu.sync_copy(ixb.at[pl.ds(0, _RK)], gidx.at[pl.ds(wid * _RK, _RK)])


def _knn_call(xp, yp, zp, sp, qx, qy, qz, qs):
    return pl.kernel(
        _knn_body,
        out_type=jax.ShapeDtypeStruct((_B * _N1 * _K,), jnp.int32),
        mesh=plsc.VectorSubcoreMesh(core_axis_name="c", subcore_axis_name="s"),
        compiler_params=pltpu.CompilerParams(needs_layout_passes=False),
        scratch_types=[
            pltpu.VMEM((_N2,), jnp.float32),
            pltpu.VMEM((_N2,), jnp.float32),
            pltpu.VMEM((_N2,), jnp.float32),
            pltpu.VMEM((_N2,), jnp.float32),
            pltpu.VMEM((_RPW,), jnp.float32),
            pltpu.VMEM((_RPW,), jnp.float32),
            pltpu.VMEM((_RPW,), jnp.float32),
            pltpu.VMEM((_RPW,), jnp.float32),
            pltpu.VMEM((_N2,), jnp.float32),
            pltpu.VMEM((_NG,), jnp.float32),
            pltpu.VMEM((_NS,), jnp.float32),
            pltpu.VMEM((_RK + 16,), jnp.int32),
        ],
    )(xp, yp, zp, sp, qx, qy, qz, qs)


def _gather_body(feat, gidx, out, idxb, rowb0, rowb1, sem0, sem1):
    wid = lax.axis_index("s") * 2 + lax.axis_index("c")
    nchunks = _RK // 128
    pltpu.sync_copy(gidx.at[pl.ds(wid * _RK, _RK)], idxb)
    bufs = (rowb0, rowb1)
    sems = (sem0, sem1)
    pltpu.async_copy(feat.at[idxb.at[pl.ds(0, 128)]], rowb0, sem0)

    def cbody(c, _):
        for par in range(2):
            @pl.when((c % 2) == par)
            def _(par=par):
                nxt = feat.at[idxb.at[pl.ds((c + 1) * 128, 128)]]
                @pl.when(c + 1 < nchunks)
                def _():
                    pltpu.async_copy(nxt, bufs[1 - par], sems[1 - par])
                pltpu.make_async_copy(nxt, bufs[par], sems[par]).wait()
                pltpu.sync_copy(bufs[par],
                                out.at[pl.ds(wid * _RK + c * 128, 128)])
        return 0

    lax.fori_loop(0, nchunks, cbody, 0)


def _gather_call(featflat, gidx):
    return pl.kernel(
        _gather_body,
        out_type=jax.ShapeDtypeStruct((_B * _N1 * _K, _C), jnp.float32),
        mesh=plsc.VectorSubcoreMesh(core_axis_name="c", subcore_axis_name="s"),
        compiler_params=pltpu.CompilerParams(needs_layout_passes=False),
        scratch_types=[
            pltpu.VMEM((_RK,), jnp.int32),
            pltpu.VMEM((128, _C), jnp.float32),
            pltpu.VMEM((128, _C), jnp.float32),
            pltpu.SemaphoreType.DMA,
            pltpu.SemaphoreType.DMA,
        ],
    )(featflat, gidx)


def _fold_stats(s_acc, m_acc, w, wt, bias, gamma, beta):
    mu = s_acc / _NTOT
    mean = jnp.dot(mu, wt, preferred_element_type=jnp.float32) + bias
    t = jnp.dot(w, m_acc / _NTOT, preferred_element_type=jnp.float32)
    ey2 = jnp.sum(t * w, axis=1)[None, :]
    ey2 = ey2 + 2.0 * bias * mean - bias * bias
    var = ey2 - mean * mean
    sc = gamma * lax.rsqrt(var + _EPS)
    at = wt * sc
    c = sc * (bias - mean) + beta
    return at, c


def _passA_body(g_ref, w_ref, wt_ref, b_ref, ga_ref, be_ref,
                at_ref, c_ref, s_acc, m_acc):
    @pl.when(pl.program_id(0) == 0)
    def _():
        s_acc[...] = jnp.zeros_like(s_acc)
        m_acc[...] = jnp.zeros_like(m_acc)

    g = g_ref[...]
    s_acc[...] += jnp.sum(g, axis=0, keepdims=True)
    m_acc[...] += lax.dot_general(g, g, (((0,), (0,)), ((), ())),
                                  preferred_element_type=jnp.float32)

    @pl.when(pl.program_id(0) == _NBLK - 1)
    def _():
        at, c = _fold_stats(s_acc[...], m_acc[...], w_ref[...], wt_ref[...],
                            b_ref[...], ga_ref[...], be_ref[...])
        at_ref[...] = at
        c_ref[...] = c


def _passB_body(g_ref, a1t_ref, c1_ref, w_ref, wt_ref, b_ref, ga_ref, be_ref,
                at_ref, c_ref, s_acc, m_acc):
    @pl.when(pl.program_id(0) == 0)
    def _():
        s_acc[...] = jnp.zeros_like(s_acc)
        m_acc[...] = jnp.zeros_like(m_acc)

    g = g_ref[...]
    r = jnp.maximum(jnp.dot(g, a1t_ref[...],
                            preferred_element_type=jnp.float32)
                    + c1_ref[...], 0.0)
    s_acc[...] += jnp.sum(r, axis=0, keepdims=True)
    m_acc[...] += lax.dot_general(r, r, (((0,), (0,)), ((), ())),
                                  preferred_element_type=jnp.float32)

    @pl.when(pl.program_id(0) == _NBLK - 1)
    def _():
        at, c = _fold_stats(s_acc[...], m_acc[...], w_ref[...], wt_ref[...],
                            b_ref[...], ga_ref[...], be_ref[...])
        at_ref[...] = at
        c_ref[...] = c


def _passC_body(g_ref, a1t_ref, c1_ref, a2t_ref, c2_ref, o_ref):
    g = g_ref[...]
    r = jnp.maximum(jnp.dot(g, a1t_ref[...],
                            preferred_element_type=jnp.float32)
                    + c1_ref[...], 0.0)
    o_ref[...] = (jnp.dot(r, a2t_ref[...], preferred_element_type=jnp.float32)
                  + c2_ref[...])


_full_spec = pl.BlockSpec((128, 128), lambda i: (0, 0))
_vec_spec = pl.BlockSpec((1, 128), lambda i: (0, 0))
_row_spec = pl.BlockSpec((_ROWBLK, 128), lambda i: (i, 0))


def _passA(G, w, wt, bias, gamma, beta):
    return pl.pallas_call(
        _passA_body,
        grid=(_NBLK,),
        in_specs=[_row_spec, _full_spec, _full_spec, _vec_spec, _vec_spec,
                  _vec_spec],
        out_specs=[_full_spec, _vec_spec],
        out_shape=[jax.ShapeDtypeStruct((128, 128), jnp.float32),
                   jax.ShapeDtypeStruct((1, 128), jnp.float32)],
        scratch_shapes=[pltpu.VMEM((1, 128), jnp.float32),
                        pltpu.VMEM((128, 128), jnp.float32)],
    )(G, w, wt, bias, gamma, beta)


def _passB(G, a1t, c1, w, wt, bias, gamma, beta):
    return pl.pallas_call(
        _passB_body,
        grid=(_NBLK,),
        in_specs=[_row_spec, _full_spec, _vec_spec, _full_spec, _full_spec,
                  _vec_spec, _vec_spec, _vec_spec],
        out_specs=[_full_spec, _vec_spec],
        out_shape=[jax.ShapeDtypeStruct((128, 128), jnp.float32),
                   jax.ShapeDtypeStruct((1, 128), jnp.float32)],
        scratch_shapes=[pltpu.VMEM((1, 128), jnp.float32),
                        pltpu.VMEM((128, 128), jnp.float32)],
    )(G, a1t, c1, w, wt, bias, gamma, beta)


def _passC(G, a1t, c1, a2t, c2):
    return pl.pallas_call(
        _passC_body,
        grid=(_NBLK,),
        in_specs=[_row_spec, _full_spec, _vec_spec, _full_spec, _vec_spec],
        out_specs=_row_spec,
        out_shape=jax.ShapeDtypeStruct((_B * _N1 * _K, _C), jnp.float32),
    )(G, a1t, c1, a2t, c2)


def _bf16_round(x):
    u = lax.bitcast_convert_type(x, jnp.uint32)
    u = (u + jnp.uint32(0x7FFF) + ((u >> 16) & jnp.uint32(1))) \
        & jnp.uint32(0xFFFF0000)
    return lax.bitcast_convert_type(u, jnp.float32)


def kernel(xyz, p1, feat, W1, b1, g1, be1, W2, b2, g2, be2):
    xyz_b = _bf16_round(xyz)
    p1_b = _bf16_round(p1)

    def _perm(a):
        return a.reshape(_B, 64, 16, 16).transpose(0, 3, 2, 1).reshape(_B, _N2)

    xp = _perm(xyz_b[:, :, 0])
    yp = _perm(xyz_b[:, :, 1])
    zp = _perm(xyz_b[:, :, 2])
    sp = _perm(jnp.sum(xyz * xyz, axis=-1))
    qx = p1_b[:, :, 0]
    qy = p1_b[:, :, 1]
    qz = p1_b[:, :, 2]
    qs = jnp.sum(p1 * p1, axis=-1)

    gidx = _knn_call(xp, yp, zp, sp, qx, qy, qz, qs)
    featflat = feat.reshape(_B * _N2, _C)
    G = _gather_call(featflat, gidx)

    w1t = W1.T
    w2t = W2.T
    b1r, g1r, be1r = b1[None, :], g1[None, :], be1[None, :]
    b2r, g2r, be2r = b2[None, :], g2[None, :], be2[None, :]
    a1t, c1 = _passA(G, W1, w1t, b1r, g1r, be1r)
    a2t, c2 = _passB(G, a1t, c1, W2, w2t, b2r, g2r, be2r)
    out = _passC(G, a1t, c1, a2t, c2)
    return out.reshape(_B, _N1, _K, _C)
